# baseline (speedup 1.0000x reference)
import functools

import jax
import jax.numpy as jnp
from jax import lax
from jax.experimental import pallas as pl
from jax.experimental.pallas import tpu as pltpu
from jax.experimental.pallas import tpu_sc as plsc

_N = 10000
_E = 160000
_NP = 10112
_CHUNK = 512
_NTILES = 32
_CHUNKS_PER_PAIR = 20
_E_PAD = 16 * _CHUNKS_PER_PAIR * _CHUNK
_RPT = _NP // 16


def _make_sc_scatter(feat, chunks_c1):
    chunks_c0 = _CHUNKS_PER_PAIR - chunks_c1
    mesh = plsc.VectorSubcoreMesh(core_axis_name="c", subcore_axis_name="s")

    @functools.partial(
        pl.kernel,
        out_type=jax.ShapeDtypeStruct((2, _NP, feat), jnp.float32),
        mesh=mesh,
        scratch_types=[
            pltpu.VMEM((2, max(chunks_c0, chunks_c1), _CHUNK), jnp.int32),
            pltpu.VMEM((2, _CHUNK, feat), jnp.float32),
            pltpu.VMEM_SHARED((_NP, feat), jnp.float32),
            pltpu.SemaphoreType.DMA,
            pltpu.SemaphoreType.DMA,
            pltpu.SemaphoreType.DMA,
        ],
        compiler_params=pltpu.CompilerParams(use_tc_tiling_on_sc=False),
    )
    def sc_scatter(y_hbm, edges_hbm, zero_hbm, out_hbm,
                   eidx, rows, acc, gsem, ssem, msem):
        c = lax.axis_index("c")
        s = lax.axis_index("s")
        r0 = s * _RPT

        def run(nch, gc0):
            idx_cp = pltpu.async_copy(
                edges_hbm.at[:, pl.ds(gc0, nch), :],
                eidx.at[:, pl.ds(0, nch), :], msem)
            pltpu.sync_copy(zero_hbm.at[pl.ds(r0, _RPT)],
                            acc.at[pl.ds(r0, _RPT)])
            idx_cp.wait()
            plsc.subcore_barrier()

            pltpu.async_copy(y_hbm.at[eidx.at[0, 0]], rows.at[0], gsem)

            def body(j, carry):
                slot = lax.rem(j, 2)
                nslot = 1 - slot

                @pl.when(j >= 1)
                def _wait_prev_scatter():
                    pltpu.make_async_copy(
                        rows.at[nslot], acc.at[eidx.at[1, j - 1]], ssem).wait()

                @pl.when(j + 1 < nch)
                def _start_next_gather():
                    pltpu.async_copy(
                        y_hbm.at[eidx.at[0, j + 1]], rows.at[nslot], gsem)

                pltpu.make_async_copy(
                    y_hbm.at[eidx.at[0, j]], rows.at[slot], gsem).wait()
                pltpu.async_copy(
                    rows.at[slot], acc.at[eidx.at[1, j]], ssem, add=True)
                return carry

            lax.fori_loop(0, nch, body, 0)
            last = nch - 1
            pltpu.make_async_copy(
                rows.at[last % 2], acc.at[eidx.at[1, last]], ssem).wait()
            plsc.subcore_barrier()
            pltpu.sync_copy(acc.at[pl.ds(r0, _RPT)],
                            out_hbm.at[c, pl.ds(r0, _RPT)])

        @pl.when(c == 0)
        def _c0():
            run(chunks_c0, s * chunks_c0)

        @pl.when(c == 1)
        def _c1():
            run(chunks_c1, 16 * chunks_c0 + s * chunks_c1)

    return sc_scatter


def _make_sc_degree(chunks_c1):
    chunks_c0 = _CHUNKS_PER_PAIR - chunks_c1
    mesh = plsc.VectorSubcoreMesh(core_axis_name="c", subcore_axis_name="s")

    @functools.partial(
        pl.kernel,
        out_type=jax.ShapeDtypeStruct((2, _NP, 16), jnp.float32),
        mesh=mesh,
        scratch_types=[
            pltpu.VMEM((max(chunks_c0, chunks_c1), _CHUNK), jnp.int32),
            pltpu.VMEM((_CHUNK, 16), jnp.float32),
            pltpu.VMEM_SHARED((_NP, 16), jnp.float32),
            pltpu.SemaphoreType.DMA,
            pltpu.SemaphoreType.DMA,
        ],
        compiler_params=pltpu.CompilerParams(use_tc_tiling_on_sc=False),
    )
    def sc_degree(ones_hbm, edges_hbm, zero_hbm, out_hbm,
                  didx, ones, acc, ssem, msem):
        c = lax.axis_index("c")
        s = lax.axis_index("s")
        r0 = s * _RPT

        def run(nch, gc0):
            idx_cp = pltpu.async_copy(
                edges_hbm.at[1, pl.ds(gc0, nch), :],
                didx.at[pl.ds(0, nch), :], msem)
            ones_cp = pltpu.async_copy(ones_hbm, ones, msem)
            pltpu.sync_copy(zero_hbm.at[pl.ds(r0, _RPT)],
                            acc.at[pl.ds(r0, _RPT)])
            idx_cp.wait()
            ones_cp.wait()
            plsc.subcore_barrier()

            def fire(j, carry):
                pltpu.async_copy(ones, acc.at[didx.at[j]], ssem, add=True)
                return carry

            lax.fori_loop(0, nch, fire, 0)

            def drain(j, carry):
                pltpu.make_async_copy(ones, acc.at[didx.at[j]], ssem).wait()
                return carry

            lax.fori_loop(0, nch, drain, 0)
            plsc.subcore_barrier()
            pltpu.sync_copy(acc.at[pl.ds(r0, _RPT)],
                            out_hbm.at[c, pl.ds(r0, _RPT)])

        @pl.when(c == 0)
        def _c0():
            run(chunks_c0, s * chunks_c0)

        @pl.when(c == 1)
        def _c1():
            run(chunks_c1, 16 * chunks_c0 + s * chunks_c1)

    return sc_degree


_sc_degree = _make_sc_degree(10)
_sc_scatter16 = _make_sc_scatter(16, 9)
_sc_scatter32 = _make_sc_scatter(32, 8)


def _proj_body(x_ref, w_ref, u0_ref):
    u0_ref[...] = jnp.dot(x_ref[...], w_ref[...],
                          preferred_element_type=jnp.float32)


def _prep_body(cnt_ref, u0_ref, dinv_ref, z0_ref):
    cnt = cnt_ref[0, :, 0:1] + cnt_ref[1, :, 0:1]
    dinv = lax.rsqrt(cnt + 1.0)
    dinv_ref[...] = dinv
    z0_ref[...] = u0_ref[...] * dinv


def _mid_body(p_ref, z_ref, dinv_ref, b_ref, w_ref, o_ref):
    dinv = dinv_ref[...]
    t = jnp.maximum(dinv * (p_ref[0] + p_ref[1] + z_ref[...]) + b_ref[...],
                    0.0)
    o_ref[...] = jnp.dot(dinv * t, w_ref[...],
                         preferred_element_type=jnp.float32)


def _last_body(p_ref, z_ref, dinv_ref, b_ref, o_ref):
    dinv = dinv_ref[...]
    t = jnp.maximum(dinv * (p_ref[0] + p_ref[1] + z_ref[...]) + b_ref[...],
                    0.0)
    o_ref[...] = dinv * t


def _final_body(p_ref, y_ref, dinv_ref, w_ref, b_ref, o_ref):
    agg = dinv_ref[...] * (p_ref[0] + p_ref[1] + y_ref[...])
    o_ref[...] = jnp.dot(agg, w_ref[...],
                         preferred_element_type=jnp.float32) + b_ref[...]


_BR = 2000


def _proj_call(x, w1):
    return pl.pallas_call(
        _proj_body,
        grid=(_N // _BR,),
        in_specs=[
            pl.BlockSpec((_BR, 128), lambda i: (i, 0)),
            pl.BlockSpec((128, 32), lambda i: (0, 0)),
        ],
        out_specs=pl.BlockSpec((_BR, 32), lambda i: (i, 0)),
        out_shape=jax.ShapeDtypeStruct((_N, 32), jnp.float32),
    )(x, w1)


def _prep_call(cnt, u0):
    return pl.pallas_call(
        _prep_body,
        grid=(_N // _BR,),
        in_specs=[
            pl.BlockSpec((2, _BR, 16), lambda i: (0, i, 0)),
            pl.BlockSpec((_BR, 32), lambda i: (i, 0)),
        ],
        out_specs=[
            pl.BlockSpec((_BR, 1), lambda i: (i, 0)),
            pl.BlockSpec((_BR, 32), lambda i: (i, 0)),
        ],
        out_shape=[
            jax.ShapeDtypeStruct((_N, 1), jnp.float32),
            jax.ShapeDtypeStruct((_N, 32), jnp.float32),
        ],
    )(cnt, u0)


def _mid_call(p, z, dinv, b, w, f_in, f_out):
    return pl.pallas_call(
        _mid_body,
        grid=(_N // _BR,),
        in_specs=[
            pl.BlockSpec((2, _BR, f_in), lambda i: (0, i, 0)),
            pl.BlockSpec((_BR, f_in), lambda i: (i, 0)),
            pl.BlockSpec((_BR, 1), lambda i: (i, 0)),
            pl.BlockSpec((1, f_in), lambda i: (0, 0)),
            pl.BlockSpec((f_in, f_out), lambda i: (0, 0)),
        ],
        out_specs=pl.BlockSpec((_BR, f_out), lambda i: (i, 0)),
        out_shape=jax.ShapeDtypeStruct((_N, f_out), jnp.float32),
    )(p, z, dinv, b.reshape(1, f_in), w)


def _last_call(p, z, dinv, b, f_in):
    return pl.pallas_call(
        _last_body,
        grid=(_N // _BR,),
        in_specs=[
            pl.BlockSpec((2, _BR, f_in), lambda i: (0, i, 0)),
            pl.BlockSpec((_BR, f_in), lambda i: (i, 0)),
            pl.BlockSpec((_BR, 1), lambda i: (i, 0)),
            pl.BlockSpec((1, f_in), lambda i: (0, 0)),
        ],
        out_specs=pl.BlockSpec((_BR, f_in), lambda i: (i, 0)),
        out_shape=jax.ShapeDtypeStruct((_N, f_in), jnp.float32),
    )(p, z, dinv, b.reshape(1, f_in))


_FR, _FC = 1000, 2048


def _final_call(p, y, dinv, w, b):
    return pl.pallas_call(
        _final_body,
        grid=(_N // _FR, pl.cdiv(_N, _FC)),
        in_specs=[
            pl.BlockSpec((2, _FR, 16), lambda i, j: (0, i, 0)),
            pl.BlockSpec((_FR, 16), lambda i, j: (i, 0)),
            pl.BlockSpec((_FR, 1), lambda i, j: (i, 0)),
            pl.BlockSpec((16, _FC), lambda i, j: (0, j)),
            pl.BlockSpec((1, _FC), lambda i, j: (0, j)),
        ],
        out_specs=pl.BlockSpec((_FR, _FC), lambda i, j: (i, j)),
        out_shape=jax.ShapeDtypeStruct((_N, _N), jnp.float32),
    )(p, y, dinv, w, b.reshape(1, _N))


def kernel(x, edge_index, W1, b1, W2, b2, W3, b3):
    ei = edge_index.astype(jnp.int32)
    pad = _E_PAD - _E
    pad_vals = jnp.stack([jnp.zeros((pad,), jnp.int32),
                          _N + (jnp.arange(pad, dtype=jnp.int32) % (_NP - _N))])
    edges = jnp.concatenate([ei, pad_vals], axis=1)
    edges = edges.reshape(2, _E_PAD // _CHUNK, _CHUNK)

    ones16 = jnp.ones((_CHUNK, 16), jnp.float32)
    z16 = jnp.zeros((_NP, 16), jnp.float32)
    z32 = jnp.zeros((_NP, 32), jnp.float32)

    u0 = _proj_call(x, W1)
    cnt = _sc_degree(ones16, edges, z16)
    dinv, z0 = _prep_call(cnt, u0)
    s0 = _sc_scatter32(z0, edges, z32)
    z1 = _mid_call(s0, z0, dinv, b1, W2, 32, 16)
    s1 = _sc_scatter16(z1, edges, z16)
    z2 = _last_call(s1, z1, dinv, b2, 16)
    s2 = _sc_scatter16(z2, edges, z16)
    return _final_call(s2, z2, dinv, W3, b3)


# speedup vs baseline: 1.0994x; 1.0994x over previous
import functools

import jax
import jax.numpy as jnp
from jax import lax
from jax.experimental import pallas as pl
from jax.experimental.pallas import tpu as pltpu
from jax.experimental.pallas import tpu_sc as plsc

_N = 10000
_E = 160000
_NP = 10112
_CHUNK = 128
_NTILES = 32
_CHUNKS_PER_PAIR = 79
_E_PAD = 16 * _CHUNKS_PER_PAIR * _CHUNK
_RPT = _NP // 16


def _make_sc_scatter(feat, chunks_c1):
    chunks_c0 = _CHUNKS_PER_PAIR - chunks_c1
    mesh = plsc.VectorSubcoreMesh(core_axis_name="c", subcore_axis_name="s")

    @functools.partial(
        pl.kernel,
        out_type=jax.ShapeDtypeStruct((2, _NP, feat), jnp.float32),
        mesh=mesh,
        scratch_types=[
            pltpu.VMEM((2, max(chunks_c0, chunks_c1), _CHUNK), jnp.int32),
            pltpu.VMEM((2, _CHUNK, feat), jnp.float32),
            pltpu.VMEM_SHARED((_NP, feat), jnp.float32),
            pltpu.SemaphoreType.DMA,
            pltpu.SemaphoreType.DMA,
            pltpu.SemaphoreType.DMA,
        ],
        compiler_params=pltpu.CompilerParams(use_tc_tiling_on_sc=False),
    )
    def sc_scatter(y_hbm, edges_hbm, zero_hbm, out_hbm,
                   eidx, rows, acc, gsem, ssem, msem):
        c = lax.axis_index("c")
        s = lax.axis_index("s")
        r0 = s * _RPT

        def run(nch, gc0):
            idx_cp = pltpu.async_copy(
                edges_hbm.at[:, pl.ds(gc0, nch), :],
                eidx.at[:, pl.ds(0, nch), :], msem)
            pltpu.sync_copy(zero_hbm.at[pl.ds(r0, _RPT)],
                            acc.at[pl.ds(r0, _RPT)])
            idx_cp.wait()
            plsc.subcore_barrier()

            pltpu.async_copy(y_hbm.at[eidx.at[0, 0]], rows.at[0], gsem)

            def body(j, carry):
                slot = lax.rem(j, 2)
                nslot = 1 - slot

                @pl.when(j >= 1)
                def _wait_prev_scatter():
                    pltpu.make_async_copy(
                        rows.at[nslot], acc.at[eidx.at[1, j - 1]], ssem).wait()

                @pl.when(j + 1 < nch)
                def _start_next_gather():
                    pltpu.async_copy(
                        y_hbm.at[eidx.at[0, j + 1]], rows.at[nslot], gsem)

                pltpu.make_async_copy(
                    y_hbm.at[eidx.at[0, j]], rows.at[slot], gsem).wait()
                pltpu.async_copy(
                    rows.at[slot], acc.at[eidx.at[1, j]], ssem, add=True)
                return carry

            lax.fori_loop(0, nch, body, 0)
            last = nch - 1
            pltpu.make_async_copy(
                rows.at[last % 2], acc.at[eidx.at[1, last]], ssem).wait()
            plsc.subcore_barrier()
            pltpu.sync_copy(acc.at[pl.ds(r0, _RPT)],
                            out_hbm.at[c, pl.ds(r0, _RPT)])

        @pl.when(c == 0)
        def _c0():
            run(chunks_c0, s * chunks_c0)

        @pl.when(c == 1)
        def _c1():
            run(chunks_c1, 16 * chunks_c0 + s * chunks_c1)

    return sc_scatter


def _make_sc_degree(chunks_c1):
    chunks_c0 = _CHUNKS_PER_PAIR - chunks_c1
    mesh = plsc.VectorSubcoreMesh(core_axis_name="c", subcore_axis_name="s")

    @functools.partial(
        pl.kernel,
        out_type=jax.ShapeDtypeStruct((2, _NP, 16), jnp.float32),
        mesh=mesh,
        scratch_types=[
            pltpu.VMEM((max(chunks_c0, chunks_c1), _CHUNK), jnp.int32),
            pltpu.VMEM((_CHUNK, 16), jnp.float32),
            pltpu.VMEM_SHARED((_NP, 16), jnp.float32),
            pltpu.SemaphoreType.DMA,
            pltpu.SemaphoreType.DMA,
        ],
        compiler_params=pltpu.CompilerParams(use_tc_tiling_on_sc=False),
    )
    def sc_degree(ones_hbm, edges_hbm, zero_hbm, out_hbm,
                  didx, ones, acc, ssem, msem):
        c = lax.axis_index("c")
        s = lax.axis_index("s")
        r0 = s * _RPT

        def run(nch, gc0):
            idx_cp = pltpu.async_copy(
                edges_hbm.at[1, pl.ds(gc0, nch), :],
                didx.at[pl.ds(0, nch), :], msem)
            ones_cp = pltpu.async_copy(ones_hbm, ones, msem)
            pltpu.sync_copy(zero_hbm.at[pl.ds(r0, _RPT)],
                            acc.at[pl.ds(r0, _RPT)])
            idx_cp.wait()
            ones_cp.wait()
            plsc.subcore_barrier()

            def fire(j, carry):
                pltpu.async_copy(ones, acc.at[didx.at[j]], ssem, add=True)
                return carry

            lax.fori_loop(0, nch, fire, 0)

            def drain(j, carry):
                pltpu.make_async_copy(ones, acc.at[didx.at[j]], ssem).wait()
                return carry

            lax.fori_loop(0, nch, drain, 0)
            plsc.subcore_barrier()
            pltpu.sync_copy(acc.at[pl.ds(r0, _RPT)],
                            out_hbm.at[c, pl.ds(r0, _RPT)])

        @pl.when(c == 0)
        def _c0():
            run(chunks_c0, s * chunks_c0)

        @pl.when(c == 1)
        def _c1():
            run(chunks_c1, 16 * chunks_c0 + s * chunks_c1)

    return sc_degree


_sc_degree = _make_sc_degree(39)
_sc_scatter16 = _make_sc_scatter(16, 33)
_sc_scatter32 = _make_sc_scatter(32, 27)


def _proj_body(x_ref, w_ref, u0_ref):
    u0_ref[...] = jnp.dot(x_ref[...], w_ref[...],
                          preferred_element_type=jnp.float32)


def _prep_body(cnt_ref, u0_ref, dinv_ref, z0_ref):
    cnt = cnt_ref[0, :, 0:1] + cnt_ref[1, :, 0:1]
    dinv = lax.rsqrt(cnt + 1.0)
    dinv_ref[...] = dinv
    z0_ref[...] = u0_ref[...] * dinv


def _mid_body(p_ref, z_ref, dinv_ref, b_ref, w_ref, o_ref):
    dinv = dinv_ref[...]
    t = jnp.maximum(dinv * (p_ref[0] + p_ref[1] + z_ref[...]) + b_ref[...],
                    0.0)
    o_ref[...] = jnp.dot(dinv * t, w_ref[...],
                         preferred_element_type=jnp.float32)


def _last_body(p_ref, z_ref, dinv_ref, b_ref, o_ref):
    dinv = dinv_ref[...]
    t = jnp.maximum(dinv * (p_ref[0] + p_ref[1] + z_ref[...]) + b_ref[...],
                    0.0)
    o_ref[...] = dinv * t


def _final_body(p_ref, y_ref, dinv_ref, w_ref, b_ref, o_ref):
    agg = dinv_ref[...] * (p_ref[0] + p_ref[1] + y_ref[...])
    o_ref[...] = jnp.dot(agg, w_ref[...],
                         preferred_element_type=jnp.float32) + b_ref[...]


_BR = 2000


def _proj_call(x, w1):
    return pl.pallas_call(
        _proj_body,
        grid=(_N // _BR,),
        in_specs=[
            pl.BlockSpec((_BR, 128), lambda i: (i, 0)),
            pl.BlockSpec((128, 32), lambda i: (0, 0)),
        ],
        out_specs=pl.BlockSpec((_BR, 32), lambda i: (i, 0)),
        out_shape=jax.ShapeDtypeStruct((_N, 32), jnp.float32),
    )(x, w1)


def _prep_call(cnt, u0):
    return pl.pallas_call(
        _prep_body,
        grid=(_N // _BR,),
        in_specs=[
            pl.BlockSpec((2, _BR, 16), lambda i: (0, i, 0)),
            pl.BlockSpec((_BR, 32), lambda i: (i, 0)),
        ],
        out_specs=[
            pl.BlockSpec((_BR, 1), lambda i: (i, 0)),
            pl.BlockSpec((_BR, 32), lambda i: (i, 0)),
        ],
        out_shape=[
            jax.ShapeDtypeStruct((_N, 1), jnp.float32),
            jax.ShapeDtypeStruct((_N, 32), jnp.float32),
        ],
    )(cnt, u0)


def _mid_call(p, z, dinv, b, w, f_in, f_out):
    return pl.pallas_call(
        _mid_body,
        grid=(_N // _BR,),
        in_specs=[
            pl.BlockSpec((2, _BR, f_in), lambda i: (0, i, 0)),
            pl.BlockSpec((_BR, f_in), lambda i: (i, 0)),
            pl.BlockSpec((_BR, 1), lambda i: (i, 0)),
            pl.BlockSpec((1, f_in), lambda i: (0, 0)),
            pl.BlockSpec((f_in, f_out), lambda i: (0, 0)),
        ],
        out_specs=pl.BlockSpec((_BR, f_out), lambda i: (i, 0)),
        out_shape=jax.ShapeDtypeStruct((_N, f_out), jnp.float32),
    )(p, z, dinv, b.reshape(1, f_in), w)


def _last_call(p, z, dinv, b, f_in):
    return pl.pallas_call(
        _last_body,
        grid=(_N // _BR,),
        in_specs=[
            pl.BlockSpec((2, _BR, f_in), lambda i: (0, i, 0)),
            pl.BlockSpec((_BR, f_in), lambda i: (i, 0)),
            pl.BlockSpec((_BR, 1), lambda i: (i, 0)),
            pl.BlockSpec((1, f_in), lambda i: (0, 0)),
        ],
        out_specs=pl.BlockSpec((_BR, f_in), lambda i: (i, 0)),
        out_shape=jax.ShapeDtypeStruct((_N, f_in), jnp.float32),
    )(p, z, dinv, b.reshape(1, f_in))


_FR, _FC = 1000, 2048


def _final_call(p, y, dinv, w, b):
    return pl.pallas_call(
        _final_body,
        grid=(_N // _FR, pl.cdiv(_N, _FC)),
        in_specs=[
            pl.BlockSpec((2, _FR, 16), lambda i, j: (0, i, 0)),
            pl.BlockSpec((_FR, 16), lambda i, j: (i, 0)),
            pl.BlockSpec((_FR, 1), lambda i, j: (i, 0)),
            pl.BlockSpec((16, _FC), lambda i, j: (0, j)),
            pl.BlockSpec((1, _FC), lambda i, j: (0, j)),
        ],
        out_specs=pl.BlockSpec((_FR, _FC), lambda i, j: (i, j)),
        out_shape=jax.ShapeDtypeStruct((_N, _N), jnp.float32),
    )(p, y, dinv, w, b.reshape(1, _N))


def kernel(x, edge_index, W1, b1, W2, b2, W3, b3):
    ei = edge_index.astype(jnp.int32)
    pad = _E_PAD - _E
    pad_vals = jnp.stack([jnp.zeros((pad,), jnp.int32),
                          _N + (jnp.arange(pad, dtype=jnp.int32) % (_NP - _N))])
    edges = jnp.concatenate([ei, pad_vals], axis=1)
    edges = edges.reshape(2, _E_PAD // _CHUNK, _CHUNK)

    ones16 = jnp.ones((_CHUNK, 16), jnp.float32)
    z16 = jnp.zeros((_NP, 16), jnp.float32)
    z32 = jnp.zeros((_NP, 32), jnp.float32)

    u0 = _proj_call(x, W1)
    cnt = _sc_degree(ones16, edges, z16)
    dinv, z0 = _prep_call(cnt, u0)
    s0 = _sc_scatter32(z0, edges, z32)
    z1 = _mid_call(s0, z0, dinv, b1, W2, 32, 16)
    s1 = _sc_scatter16(z1, edges, z16)
    z2 = _last_call(s1, z1, dinv, b2, 16)
    s2 = _sc_scatter16(z2, edges, z16)
    return _final_call(s2, z2, dinv, W3, b3)


# speedup vs baseline: 1.1285x; 1.0264x over previous
import functools

import jax
import jax.numpy as jnp
from jax import lax
from jax.experimental import pallas as pl
from jax.experimental.pallas import tpu as pltpu
from jax.experimental.pallas import tpu_sc as plsc

_N = 10000
_E = 160000
_NP = 10112
_CHUNK = 128
_NTILES = 32
_CHUNKS_PER_PAIR = 79
_E_PAD = 16 * _CHUNKS_PER_PAIR * _CHUNK
_RPT = _NP // 16


def _make_sc_scatter(feat, chunks_c1):
    chunks_c0 = _CHUNKS_PER_PAIR - chunks_c1
    mesh = plsc.VectorSubcoreMesh(core_axis_name="c", subcore_axis_name="s")

    @functools.partial(
        pl.kernel,
        out_type=jax.ShapeDtypeStruct((2, _NP, feat), jnp.float32),
        mesh=mesh,
        scratch_types=[
            pltpu.VMEM((2, max(chunks_c0, chunks_c1), _CHUNK), jnp.int32),
            pltpu.VMEM((2, _CHUNK, feat), jnp.float32),
            pltpu.VMEM_SHARED((_NP, feat), jnp.float32),
            pltpu.SemaphoreType.DMA,
            pltpu.SemaphoreType.DMA,
            pltpu.SemaphoreType.DMA,
        ],
        compiler_params=pltpu.CompilerParams(use_tc_tiling_on_sc=False),
    )
    def sc_scatter(y_hbm, edges_hbm, zero_hbm, out_hbm,
                   eidx, rows, acc, gsem, ssem, msem):
        c = lax.axis_index("c")
        s = lax.axis_index("s")
        r0 = s * _RPT

        def run(nch, gc0):
            idx_cp = pltpu.async_copy(
                edges_hbm.at[:, pl.ds(gc0, nch), :],
                eidx.at[:, pl.ds(0, nch), :], msem)
            pltpu.sync_copy(zero_hbm.at[pl.ds(r0, _RPT)],
                            acc.at[pl.ds(r0, _RPT)])
            idx_cp.wait()
            plsc.subcore_barrier()

            pltpu.async_copy(y_hbm.at[eidx.at[0, 0]], rows.at[0], gsem)

            def body(j, carry):
                slot = lax.rem(j, 2)
                nslot = 1 - slot

                @pl.when(j >= 1)
                def _wait_prev_scatter():
                    pltpu.make_async_copy(
                        rows.at[nslot], acc.at[eidx.at[1, j - 1]], ssem).wait()

                @pl.when(j + 1 < nch)
                def _start_next_gather():
                    pltpu.async_copy(
                        y_hbm.at[eidx.at[0, j + 1]], rows.at[nslot], gsem)

                pltpu.make_async_copy(
                    y_hbm.at[eidx.at[0, j]], rows.at[slot], gsem).wait()
                pltpu.async_copy(
                    rows.at[slot], acc.at[eidx.at[1, j]], ssem, add=True)
                return carry

            lax.fori_loop(0, nch, body, 0)
            last = nch - 1
            pltpu.make_async_copy(
                rows.at[last % 2], acc.at[eidx.at[1, last]], ssem).wait()
            plsc.subcore_barrier()
            pltpu.sync_copy(acc.at[pl.ds(r0, _RPT)],
                            out_hbm.at[c, pl.ds(r0, _RPT)])

        @pl.when(c == 0)
        def _c0():
            run(chunks_c0, s * chunks_c0)

        @pl.when(c == 1)
        def _c1():
            run(chunks_c1, 16 * chunks_c0 + s * chunks_c1)

    return sc_scatter


def _make_sc_degree(chunks_c1):
    chunks_c0 = _CHUNKS_PER_PAIR - chunks_c1
    mesh = plsc.VectorSubcoreMesh(core_axis_name="c", subcore_axis_name="s")

    @functools.partial(
        pl.kernel,
        out_type=jax.ShapeDtypeStruct((2, _NP, 16), jnp.float32),
        mesh=mesh,
        scratch_types=[
            pltpu.VMEM((max(chunks_c0, chunks_c1), _CHUNK), jnp.int32),
            pltpu.VMEM((_CHUNK, 16), jnp.float32),
            pltpu.VMEM_SHARED((_NP, 16), jnp.float32),
            pltpu.SemaphoreType.DMA,
            pltpu.SemaphoreType.DMA,
        ],
        compiler_params=pltpu.CompilerParams(use_tc_tiling_on_sc=False),
    )
    def sc_degree(ones_hbm, edges_hbm, zero_hbm, out_hbm,
                  didx, ones, acc, ssem, msem):
        c = lax.axis_index("c")
        s = lax.axis_index("s")
        r0 = s * _RPT

        def run(nch, gc0):
            idx_cp = pltpu.async_copy(
                edges_hbm.at[1, pl.ds(gc0, nch), :],
                didx.at[pl.ds(0, nch), :], msem)
            ones_cp = pltpu.async_copy(ones_hbm, ones, msem)
            pltpu.sync_copy(zero_hbm.at[pl.ds(r0, _RPT)],
                            acc.at[pl.ds(r0, _RPT)])
            idx_cp.wait()
            ones_cp.wait()
            plsc.subcore_barrier()

            def fire(j, carry):
                pltpu.async_copy(ones, acc.at[didx.at[j]], ssem, add=True)
                return carry

            lax.fori_loop(0, nch, fire, 0)

            def drain(j, carry):
                pltpu.make_async_copy(ones, acc.at[didx.at[j]], ssem).wait()
                return carry

            lax.fori_loop(0, nch, drain, 0)
            plsc.subcore_barrier()
            pltpu.sync_copy(acc.at[pl.ds(r0, _RPT)],
                            out_hbm.at[c, pl.ds(r0, _RPT)])

        @pl.when(c == 0)
        def _c0():
            run(chunks_c0, s * chunks_c0)

        @pl.when(c == 1)
        def _c1():
            run(chunks_c1, 16 * chunks_c0 + s * chunks_c1)

    return sc_degree


_sc_degree = _make_sc_degree(39)
_sc_scatter16 = _make_sc_scatter(16, 33)
_sc_scatter32 = _make_sc_scatter(32, 27)


def _proj_body(x_ref, w_ref, u0_ref):
    u0_ref[...] = jnp.dot(x_ref[...], w_ref[...],
                          preferred_element_type=jnp.float32)


def _prep_body(cnt_ref, u0_ref, dinv_ref, z0_ref):
    cnt = cnt_ref[0, :, 0:1] + cnt_ref[1, :, 0:1]
    dinv = lax.rsqrt(cnt + 1.0)
    dinv_ref[...] = dinv
    z0_ref[...] = u0_ref[...] * dinv


def _mid_body(p_ref, z_ref, dinv_ref, b_ref, w_ref, o_ref):
    dinv = dinv_ref[...]
    t = jnp.maximum(dinv * (p_ref[0] + p_ref[1] + z_ref[...]) + b_ref[...],
                    0.0)
    o_ref[...] = jnp.dot(dinv * t, w_ref[...],
                         preferred_element_type=jnp.float32)


def _last_body(p_ref, z_ref, dinv_ref, b_ref, o_ref):
    dinv = dinv_ref[...]
    t = jnp.maximum(dinv * (p_ref[0] + p_ref[1] + z_ref[...]) + b_ref[...],
                    0.0)
    o_ref[...] = dinv * t


def _final_body(p_ref, y_ref, dinv_ref, w_ref, b_ref, o_ref):
    agg = dinv_ref[...] * (p_ref[0] + p_ref[1] + y_ref[...])
    o_ref[...] = jnp.dot(agg, w_ref[...],
                         preferred_element_type=jnp.float32) + b_ref[...]


_BR = 2000


def _proj_call(x, w1):
    return pl.pallas_call(
        _proj_body,
        grid=(_N // _BR,),
        in_specs=[
            pl.BlockSpec((_BR, 128), lambda i: (i, 0)),
            pl.BlockSpec((128, 32), lambda i: (0, 0)),
        ],
        out_specs=pl.BlockSpec((_BR, 32), lambda i: (i, 0)),
        out_shape=jax.ShapeDtypeStruct((_N, 32), jnp.float32),
    )(x, w1)


def _prep_call(cnt, u0):
    return pl.pallas_call(
        _prep_body,
        grid=(_N // _BR,),
        in_specs=[
            pl.BlockSpec((2, _BR, 16), lambda i: (0, i, 0)),
            pl.BlockSpec((_BR, 32), lambda i: (i, 0)),
        ],
        out_specs=[
            pl.BlockSpec((_BR, 1), lambda i: (i, 0)),
            pl.BlockSpec((_BR, 32), lambda i: (i, 0)),
        ],
        out_shape=[
            jax.ShapeDtypeStruct((_N, 1), jnp.float32),
            jax.ShapeDtypeStruct((_N, 32), jnp.float32),
        ],
    )(cnt, u0)


def _mid_call(p, z, dinv, b, w, f_in, f_out):
    return pl.pallas_call(
        _mid_body,
        grid=(_N // _BR,),
        in_specs=[
            pl.BlockSpec((2, _BR, f_in), lambda i: (0, i, 0)),
            pl.BlockSpec((_BR, f_in), lambda i: (i, 0)),
            pl.BlockSpec((_BR, 1), lambda i: (i, 0)),
            pl.BlockSpec((1, f_in), lambda i: (0, 0)),
            pl.BlockSpec((f_in, f_out), lambda i: (0, 0)),
        ],
        out_specs=pl.BlockSpec((_BR, f_out), lambda i: (i, 0)),
        out_shape=jax.ShapeDtypeStruct((_N, f_out), jnp.float32),
    )(p, z, dinv, b.reshape(1, f_in), w)


def _last_call(p, z, dinv, b, f_in):
    return pl.pallas_call(
        _last_body,
        grid=(_N // _BR,),
        in_specs=[
            pl.BlockSpec((2, _BR, f_in), lambda i: (0, i, 0)),
            pl.BlockSpec((_BR, f_in), lambda i: (i, 0)),
            pl.BlockSpec((_BR, 1), lambda i: (i, 0)),
            pl.BlockSpec((1, f_in), lambda i: (0, 0)),
        ],
        out_specs=pl.BlockSpec((_BR, f_in), lambda i: (i, 0)),
        out_shape=jax.ShapeDtypeStruct((_N, f_in), jnp.float32),
    )(p, z, dinv, b.reshape(1, f_in))


_FR, _FC = 2000, 2048


def _final_call(p, y, dinv, w, b):
    return pl.pallas_call(
        _final_body,
        grid=(_N // _FR, pl.cdiv(_N, _FC)),
        in_specs=[
            pl.BlockSpec((2, _FR, 16), lambda i, j: (0, i, 0)),
            pl.BlockSpec((_FR, 16), lambda i, j: (i, 0)),
            pl.BlockSpec((_FR, 1), lambda i, j: (i, 0)),
            pl.BlockSpec((16, _FC), lambda i, j: (0, j)),
            pl.BlockSpec((1, _FC), lambda i, j: (0, j)),
        ],
        out_specs=pl.BlockSpec((_FR, _FC), lambda i, j: (i, j)),
        out_shape=jax.ShapeDtypeStruct((_N, _N), jnp.float32),
    )(p, y, dinv, w, b.reshape(1, _N))


def kernel(x, edge_index, W1, b1, W2, b2, W3, b3):
    ei = edge_index.astype(jnp.int32)
    pad = _E_PAD - _E
    pad_vals = jnp.stack([jnp.zeros((pad,), jnp.int32),
                          _N + (jnp.arange(pad, dtype=jnp.int32) % (_NP - _N))])
    edges = jnp.concatenate([ei, pad_vals], axis=1)
    edges = edges.reshape(2, _E_PAD // _CHUNK, _CHUNK)

    ones16 = jnp.ones((_CHUNK, 16), jnp.float32)
    z16 = jnp.zeros((_NP, 16), jnp.float32)
    z32 = jnp.zeros((_NP, 32), jnp.float32)

    u0 = _proj_call(x, W1)
    cnt = _sc_degree(ones16, edges, z16)
    dinv, z0 = _prep_call(cnt, u0)
    s0 = _sc_scatter32(z0, edges, z32)
    z1 = _mid_call(s0, z0, dinv, b1, W2, 32, 16)
    s1 = _sc_scatter16(z1, edges, z16)
    z2 = _last_call(s1, z1, dinv, b2, 16)
    s2 = _sc_scatter16(z2, edges, z16)
    return _final_call(s2, z2, dinv, W3, b3)
